# baseline (speedup 1.0000x reference)
import jax
import jax.numpy as jnp
from jax.experimental import pallas as pl
from jax.experimental.pallas import tpu as pltpu


def _fused_kernel(x_ref, adj_hbm, w0w_ref, w0b_ref, wrw_ref, wrb_ref,
                  lng_ref, lnb_ref, out_ref, stats_ref,
                  abuf, adjbf_ref, sem):
    b = pl.program_id(0)
    n = adjbf_ref.shape[1]
    n_rel = adjbf_ref.shape[0]
    f32 = jnp.float32

    n_b = pl.num_programs(0)
    n_slots = abuf.shape[0]

    def slot(j):
        return jax.lax.rem(b * n_rel + j, n_slots)

    def adj_copy(bi, j, s):
        return pltpu.make_async_copy(
            adj_hbm.at[bi, j], abuf.at[s], sem.at[s])

    @pl.when(b == 0)
    def _prologue():
        for j in range(n_slots):
            adj_copy(0, j, j).start()

    xb = x_ref[0].astype(jnp.bfloat16)
    ones = jnp.ones((n, 1), dtype=jnp.bfloat16)
    d = w0w_ref.shape[1]

    s1 = jnp.concatenate(
        [jnp.dot(xb, w0w_ref[0], preferred_element_type=f32) + w0b_ref[0],
         jnp.ones((n, 128), dtype=f32)], axis=1)
    ones_blk = jnp.ones((n, 128), dtype=jnp.bfloat16)
    hs = [jnp.concatenate(
        [(jnp.dot(xb, wrw_ref[j, 0], preferred_element_type=f32)
          + wrb_ref[j, 0]).astype(jnp.bfloat16), ones_blk], axis=1)
        for j in range(n_rel)]
    deg_snaps = []
    cols = []
    for j in range(n_rel):
        sj = slot(j)
        adj_copy(b, j, sj).wait()
        ab = abuf[sj].astype(jnp.bfloat16)
        adjbf_ref[j] = ab

        if j + n_slots < n_rel:
            adj_copy(b, j + n_slots, sj).start()
        else:
            jn = j + n_slots - n_rel

            @pl.when(b + 1 < n_b)
            def _prefetch_next():
                adj_copy(jnp.minimum(b + 1, n_b - 1), jn, sj).start()

        s1 = s1 + jnp.dot(ab, hs[j], preferred_element_type=f32)
        deg_snaps.append(s1[:, d:d + 1])
        cols.append(jax.lax.dot_general(
            ab, ones, (((0,), (0,)), ((), ())),
            preferred_element_type=f32))

    msk = jnp.zeros((n, 1), dtype=f32)
    prev = jnp.ones((n, 1), dtype=f32)
    for j in range(n_rel):
        row_j = deg_snaps[j] - prev
        prev = deg_snaps[j]
        msk = msk + ((row_j + cols[j]) == 0.0).astype(f32)

    den = deg_snaps[-1]
    y1 = jnp.maximum(s1[:, :d] / den, 0.0)
    x2 = y1.astype(jnp.bfloat16)

    s2 = jnp.dot(x2, w0w_ref[1], preferred_element_type=f32) + w0b_ref[1]
    h2s = [(jnp.dot(x2, wrw_ref[j, 1], preferred_element_type=f32)
            + wrb_ref[j, 1]).astype(jnp.bfloat16) for j in range(n_rel)]
    for j in range(n_rel):
        s2 = s2 + jnp.dot(adjbf_ref[j], h2s[j], preferred_element_type=f32)

    y2 = jnp.maximum(s2 / den, 0.0)
    mean = jnp.mean(y2, axis=1, keepdims=True)
    var = jnp.mean((y2 - mean) ** 2, axis=1, keepdims=True)
    yn = (y2 - mean) * jax.lax.rsqrt(var + 1e-5)
    out_ref[0] = yn * lng_ref[...] + lnb_ref[...]
    stats_ref[0] = jnp.concatenate([den, msk] + [jnp.zeros_like(den)] * 6,
                                   axis=1)


def kernel(nodes, adj, section, W0_w, W0_b, Wr_w, Wr_b, ln_g, ln_b):
    B, N, D = nodes.shape
    R = adj.shape[1]
    del section

    W0_b3 = W0_b.reshape(W0_b.shape[0], 1, D)
    Wr_b4 = Wr_b.reshape(R, Wr_b.shape[1], 1, D)
    W0_wb = W0_w.astype(jnp.bfloat16)
    Wr_wb = Wr_w.astype(jnp.bfloat16)
    ln_g2 = ln_g.reshape(1, D)
    ln_b2 = ln_b.reshape(1, D)

    L = W0_w.shape[0]
    full = lambda *shape: pl.BlockSpec(shape, lambda b: (0,) * len(shape))

    gcn2, stats = pl.pallas_call(
        _fused_kernel,
        grid=(B,),
        in_specs=[
            pl.BlockSpec((1, N, D), lambda b: (b, 0, 0)),
            pl.BlockSpec(memory_space=pltpu.MemorySpace.HBM),
            full(L, D, D),
            full(L, 1, D),
            full(R, L, D, D),
            full(R, L, 1, D),
            full(1, D),
            full(1, D),
        ],
        out_specs=[
            pl.BlockSpec((1, N, D), lambda b: (b, 0, 0)),
            pl.BlockSpec((1, N, 8), lambda b: (b, 0, 0)),
        ],
        out_shape=[
            jax.ShapeDtypeStruct((B, N, D), jnp.float32),
            jax.ShapeDtypeStruct((B, N, 8), jnp.float32),
        ],
        scratch_shapes=[
            pltpu.VMEM((3, N, N), jnp.float32),
            pltpu.VMEM((R, N, N), jnp.bfloat16),
            pltpu.SemaphoreType.DMA((3,)),
        ],
        compiler_params=pltpu.CompilerParams(
            dimension_semantics=("arbitrary",)),
    )(nodes, adj, W0_wb, W0_b3, Wr_wb, Wr_b4, ln_g2, ln_b2)

    masks = stats[:, :, 1].astype(jnp.int32)
    return gcn2, masks


# speedup vs baseline: 1.0018x; 1.0018x over previous
import jax
import jax.numpy as jnp
from jax.experimental import pallas as pl
from jax.experimental.pallas import tpu as pltpu


def _fused_kernel(x_ref, adj_hbm, w0w_ref, w0b_ref, wrw_ref, wrb_ref,
                  lng_ref, lnb_ref, out_ref, stats_ref,
                  abuf, adjbf_ref, sem):
    b = pl.program_id(0)
    n = adjbf_ref.shape[1]
    n_rel = adjbf_ref.shape[0]
    f32 = jnp.float32

    n_b = pl.num_programs(0)
    n_slots = abuf.shape[0]

    def slot(j):
        return jax.lax.rem(b * n_rel + j, n_slots)

    nh = abuf.shape[1] // 2

    def adj_copies(bi, j, s):
        return [pltpu.make_async_copy(
            adj_hbm.at[bi, j, pl.ds(h * nh, nh)],
            abuf.at[s, pl.ds(h * nh, nh)], sem.at[s, h]) for h in (0, 1)]

    def start_copies(bi, j, s):
        for c in adj_copies(bi, j, s):
            c.start()

    def wait_copies(bi, j, s):
        for c in adj_copies(bi, j, s):
            c.wait()

    @pl.when(b == 0)
    def _prologue():
        for j in range(n_slots):
            start_copies(0, j, j)

    xb = x_ref[0].astype(jnp.bfloat16)
    ones = jnp.ones((n, 1), dtype=jnp.bfloat16)
    d = w0w_ref.shape[1]

    s1 = jnp.concatenate(
        [jnp.dot(xb, w0w_ref[0], preferred_element_type=f32) + w0b_ref[0],
         jnp.ones((n, 128), dtype=f32)], axis=1)
    ones_blk = jnp.ones((n, 128), dtype=jnp.bfloat16)
    hs = [jnp.concatenate(
        [(jnp.dot(xb, wrw_ref[j, 0], preferred_element_type=f32)
          + wrb_ref[j, 0]).astype(jnp.bfloat16), ones_blk], axis=1)
        for j in range(n_rel)]
    deg_snaps = []
    cols = []
    for j in range(n_rel):
        sj = slot(j)
        wait_copies(b, j, sj)
        ab = abuf[sj].astype(jnp.bfloat16)
        adjbf_ref[j] = ab

        if j + n_slots < n_rel:
            start_copies(b, j + n_slots, sj)
        else:
            jn = j + n_slots - n_rel

            @pl.when(b + 1 < n_b)
            def _prefetch_next():
                start_copies(jnp.minimum(b + 1, n_b - 1), jn, sj)

        s1 = s1 + jnp.dot(ab, hs[j], preferred_element_type=f32)
        deg_snaps.append(s1[:, d:d + 1])
        cols.append(jax.lax.dot_general(
            ab, ones, (((0,), (0,)), ((), ())),
            preferred_element_type=f32))

    msk = jnp.zeros((n, 1), dtype=f32)
    prev = jnp.ones((n, 1), dtype=f32)
    for j in range(n_rel):
        row_j = deg_snaps[j] - prev
        prev = deg_snaps[j]
        msk = msk + ((row_j + cols[j]) == 0.0).astype(f32)

    den = deg_snaps[-1]
    y1 = jnp.maximum(s1[:, :d] / den, 0.0)
    x2 = y1.astype(jnp.bfloat16)

    s2 = jnp.dot(x2, w0w_ref[1], preferred_element_type=f32) + w0b_ref[1]
    h2s = [(jnp.dot(x2, wrw_ref[j, 1], preferred_element_type=f32)
            + wrb_ref[j, 1]).astype(jnp.bfloat16) for j in range(n_rel)]
    for j in range(n_rel):
        s2 = s2 + jnp.dot(adjbf_ref[j], h2s[j], preferred_element_type=f32)

    y2 = jnp.maximum(s2 / den, 0.0)
    mean = jnp.mean(y2, axis=1, keepdims=True)
    var = jnp.mean((y2 - mean) ** 2, axis=1, keepdims=True)
    yn = (y2 - mean) * jax.lax.rsqrt(var + 1e-5)
    out_ref[0] = yn * lng_ref[...] + lnb_ref[...]
    stats_ref[0] = jnp.concatenate([den, msk] + [jnp.zeros_like(den)] * 6,
                                   axis=1)


def kernel(nodes, adj, section, W0_w, W0_b, Wr_w, Wr_b, ln_g, ln_b):
    B, N, D = nodes.shape
    R = adj.shape[1]
    del section

    W0_b3 = W0_b.reshape(W0_b.shape[0], 1, D)
    Wr_b4 = Wr_b.reshape(R, Wr_b.shape[1], 1, D)
    W0_wb = W0_w.astype(jnp.bfloat16)
    Wr_wb = Wr_w.astype(jnp.bfloat16)
    ln_g2 = ln_g.reshape(1, D)
    ln_b2 = ln_b.reshape(1, D)

    L = W0_w.shape[0]
    full = lambda *shape: pl.BlockSpec(shape, lambda b: (0,) * len(shape))

    gcn2, stats = pl.pallas_call(
        _fused_kernel,
        grid=(B,),
        in_specs=[
            pl.BlockSpec((1, N, D), lambda b: (b, 0, 0)),
            pl.BlockSpec(memory_space=pltpu.MemorySpace.HBM),
            full(L, D, D),
            full(L, 1, D),
            full(R, L, D, D),
            full(R, L, 1, D),
            full(1, D),
            full(1, D),
        ],
        out_specs=[
            pl.BlockSpec((1, N, D), lambda b: (b, 0, 0)),
            pl.BlockSpec((1, N, 8), lambda b: (b, 0, 0)),
        ],
        out_shape=[
            jax.ShapeDtypeStruct((B, N, D), jnp.float32),
            jax.ShapeDtypeStruct((B, N, 8), jnp.float32),
        ],
        scratch_shapes=[
            pltpu.VMEM((3, N, N), jnp.float32),
            pltpu.VMEM((R, N, N), jnp.bfloat16),
            pltpu.SemaphoreType.DMA((3, 2)),
        ],
        compiler_params=pltpu.CompilerParams(
            dimension_semantics=("arbitrary",)),
    )(nodes, adj, W0_wb, W0_b3, Wr_wb, Wr_b4, ln_g2, ln_b2)

    masks = stats[:, :, 1].astype(jnp.int32)
    return gcn2, masks


# speedup vs baseline: 1.0028x; 1.0010x over previous
import jax
import jax.numpy as jnp
from jax.experimental import pallas as pl
from jax.experimental.pallas import tpu as pltpu


def _fused_kernel(x_ref, adj_hbm, w0w_ref, w0b_ref, wrw_ref, wrb_ref,
                  lng_ref, lnb_ref, out_ref, stats_ref,
                  abuf, adjbf_ref, hall_ref, sem):
    b = pl.program_id(0)
    n = adjbf_ref.shape[0]
    n_rel = adjbf_ref.shape[1] // n
    f32 = jnp.float32

    n_b = pl.num_programs(0)
    n_slots = abuf.shape[0]

    def slot(j):
        return jax.lax.rem(b * n_rel + j, n_slots)

    nh = abuf.shape[1] // 2

    def adj_copies(bi, j, s):
        return [pltpu.make_async_copy(
            adj_hbm.at[bi, j, pl.ds(h * nh, nh)],
            abuf.at[s, pl.ds(h * nh, nh)], sem.at[s, h]) for h in (0, 1)]

    def start_copies(bi, j, s):
        for c in adj_copies(bi, j, s):
            c.start()

    def wait_copies(bi, j, s):
        for c in adj_copies(bi, j, s):
            c.wait()

    @pl.when(b == 0)
    def _prologue():
        for j in range(n_slots):
            start_copies(0, j, j)

    xb = x_ref[0].astype(jnp.bfloat16)
    ones = jnp.ones((n, 1), dtype=jnp.bfloat16)
    d = w0w_ref.shape[1]

    s1 = jnp.concatenate(
        [jnp.dot(xb, w0w_ref[0], preferred_element_type=f32) + w0b_ref[0],
         jnp.ones((n, 128), dtype=f32)], axis=1)
    ones_blk = jnp.ones((n, 128), dtype=jnp.bfloat16)
    hs = [jnp.concatenate(
        [(jnp.dot(xb, wrw_ref[j, 0], preferred_element_type=f32)
          + wrb_ref[j, 0]).astype(jnp.bfloat16), ones_blk], axis=1)
        for j in range(n_rel)]
    deg_snaps = []
    cols = []
    for j in range(n_rel):
        sj = slot(j)
        wait_copies(b, j, sj)
        ab = abuf[sj].astype(jnp.bfloat16)
        adjbf_ref[:, j * n:(j + 1) * n] = ab

        if j + n_slots < n_rel:
            start_copies(b, j + n_slots, sj)
        else:
            jn = j + n_slots - n_rel

            @pl.when(b + 1 < n_b)
            def _prefetch_next():
                start_copies(jnp.minimum(b + 1, n_b - 1), jn, sj)

        s1 = s1 + jnp.dot(ab, hs[j], preferred_element_type=f32)
        deg_snaps.append(s1[:, d:d + 1])
        cols.append(jax.lax.dot_general(
            ab, ones, (((0,), (0,)), ((), ())),
            preferred_element_type=f32))

    msk = jnp.zeros((n, 1), dtype=f32)
    prev = jnp.ones((n, 1), dtype=f32)
    for j in range(n_rel):
        row_j = deg_snaps[j] - prev
        prev = deg_snaps[j]
        msk = msk + ((row_j + cols[j]) == 0.0).astype(f32)

    den = deg_snaps[-1]
    y1 = jnp.maximum(s1[:, :d] / den, 0.0)
    x2 = y1.astype(jnp.bfloat16)

    for j in range(n_rel):
        hall_ref[j * n:(j + 1) * n, :] = (
            jnp.dot(x2, wrw_ref[j, 1], preferred_element_type=f32)
            + wrb_ref[j, 1]).astype(jnp.bfloat16)
    s2 = jnp.dot(x2, w0w_ref[1], preferred_element_type=f32) + w0b_ref[1] \
        + jnp.dot(adjbf_ref[...], hall_ref[...],
                  preferred_element_type=f32)

    y2 = jnp.maximum(s2 / den, 0.0)
    mean = jnp.mean(y2, axis=1, keepdims=True)
    var = jnp.mean((y2 - mean) ** 2, axis=1, keepdims=True)
    yn = (y2 - mean) * jax.lax.rsqrt(var + 1e-5)
    out_ref[0] = yn * lng_ref[...] + lnb_ref[...]
    stats_ref[0] = jnp.concatenate([den, msk] + [jnp.zeros_like(den)] * 6,
                                   axis=1)


def kernel(nodes, adj, section, W0_w, W0_b, Wr_w, Wr_b, ln_g, ln_b):
    B, N, D = nodes.shape
    R = adj.shape[1]
    del section

    W0_b3 = W0_b.reshape(W0_b.shape[0], 1, D)
    Wr_b4 = Wr_b.reshape(R, Wr_b.shape[1], 1, D)
    W0_wb = W0_w.astype(jnp.bfloat16)
    Wr_wb = Wr_w.astype(jnp.bfloat16)
    ln_g2 = ln_g.reshape(1, D)
    ln_b2 = ln_b.reshape(1, D)

    L = W0_w.shape[0]
    full = lambda *shape: pl.BlockSpec(shape, lambda b: (0,) * len(shape))

    gcn2, stats = pl.pallas_call(
        _fused_kernel,
        grid=(B,),
        in_specs=[
            pl.BlockSpec((1, N, D), lambda b: (b, 0, 0)),
            pl.BlockSpec(memory_space=pltpu.MemorySpace.HBM),
            full(L, D, D),
            full(L, 1, D),
            full(R, L, D, D),
            full(R, L, 1, D),
            full(1, D),
            full(1, D),
        ],
        out_specs=[
            pl.BlockSpec((1, N, D), lambda b: (b, 0, 0)),
            pl.BlockSpec((1, N, 8), lambda b: (b, 0, 0)),
        ],
        out_shape=[
            jax.ShapeDtypeStruct((B, N, D), jnp.float32),
            jax.ShapeDtypeStruct((B, N, 8), jnp.float32),
        ],
        scratch_shapes=[
            pltpu.VMEM((3, N, N), jnp.float32),
            pltpu.VMEM((N, R * N), jnp.bfloat16),
            pltpu.VMEM((R * N, D), jnp.bfloat16),
            pltpu.SemaphoreType.DMA((3, 2)),
        ],
        compiler_params=pltpu.CompilerParams(
            dimension_semantics=("arbitrary",)),
    )(nodes, adj, W0_wb, W0_b3, Wr_wb, Wr_b4, ln_g2, ln_b2)

    masks = stats[:, :, 1].astype(jnp.int32)
    return gcn2, masks


# speedup vs baseline: 1.1102x; 1.1072x over previous
import jax
import jax.numpy as jnp
from jax.experimental import pallas as pl
from jax.experimental.pallas import tpu as pltpu


def _fused_kernel(x_ref, adj_hbm, w0w_ref, w0b_ref, wrw_ref, wrb_ref,
                  lng_ref, lnb_ref, out_ref, stats_ref,
                  abuf, adjbf_ref, hall_ref, sem):
    b = pl.program_id(0)
    n = adjbf_ref.shape[0]
    n_rel = adjbf_ref.shape[1] // n
    f32 = jnp.float32

    n_b = pl.num_programs(0)
    n_slots = abuf.shape[0]

    def slot(j):
        return jax.lax.rem(b * n_rel + j, n_slots)

    nh = abuf.shape[1] // 2

    def adj_copies(bi, j, s):
        return [pltpu.make_async_copy(
            adj_hbm.at[bi, j, pl.ds(h * nh, nh)],
            abuf.at[s, pl.ds(h * nh, nh)], sem.at[s, h]) for h in (0, 1)]

    def start_copies(bi, j, s):
        for c in adj_copies(bi, j, s):
            c.start()

    def wait_copies(bi, j, s):
        for c in adj_copies(bi, j, s):
            c.wait()

    @pl.when(b == 0)
    def _prologue():
        for j in range(n_slots):
            start_copies(0, j, j)

    xb = x_ref[0].astype(jnp.bfloat16)
    ones = jnp.ones((n, 1), dtype=jnp.bfloat16)
    d = w0w_ref.shape[1]

    s1 = jnp.dot(xb, w0w_ref[0], preferred_element_type=f32) + w0b_ref[0]
    hs = [(jnp.dot(xb, wrw_ref[j, 0], preferred_element_type=f32)
           + wrb_ref[j, 0]).astype(jnp.bfloat16) for j in range(n_rel)]
    den = jnp.ones((n, 1), dtype=f32)
    msk = jnp.zeros((n, 1), dtype=f32)
    for j in range(n_rel):
        sj = slot(j)
        wait_copies(b, j, sj)
        ab = abuf[sj].astype(jnp.bfloat16)
        adjbf_ref[:, j * n:(j + 1) * n] = ab

        if j + n_slots < n_rel:
            start_copies(b, j + n_slots, sj)
        else:
            jn = j + n_slots - n_rel

            @pl.when(b + 1 < n_b)
            def _prefetch_next():
                start_copies(jnp.minimum(b + 1, n_b - 1), jn, sj)

        s1 = s1 + jnp.dot(ab, hs[j], preferred_element_type=f32)
        row = jnp.sum(ab, axis=1, keepdims=True, dtype=f32)
        col = jax.lax.dot_general(ab, ones, (((0,), (0,)), ((), ())),
                                  preferred_element_type=f32)
        den = den + row
        msk = msk + ((row + col) == 0.0).astype(f32)

    y1 = jnp.maximum(s1 / den, 0.0)
    x2 = y1.astype(jnp.bfloat16)

    for j in range(n_rel):
        hall_ref[j * n:(j + 1) * n, :] = (
            jnp.dot(x2, wrw_ref[j, 1], preferred_element_type=f32)
            + wrb_ref[j, 1]).astype(jnp.bfloat16)
    s2 = jnp.dot(x2, w0w_ref[1], preferred_element_type=f32) + w0b_ref[1] \
        + jnp.dot(adjbf_ref[...], hall_ref[...],
                  preferred_element_type=f32)

    y2 = jnp.maximum(s2 / den, 0.0)
    mean = jnp.mean(y2, axis=1, keepdims=True)
    var = jnp.mean((y2 - mean) ** 2, axis=1, keepdims=True)
    yn = (y2 - mean) * jax.lax.rsqrt(var + 1e-5)
    out_ref[0] = yn * lng_ref[...] + lnb_ref[...]
    stats_ref[0] = jnp.concatenate([den, msk] + [jnp.zeros_like(den)] * 6,
                                   axis=1)


def kernel(nodes, adj, section, W0_w, W0_b, Wr_w, Wr_b, ln_g, ln_b):
    B, N, D = nodes.shape
    R = adj.shape[1]
    del section

    W0_b3 = W0_b.reshape(W0_b.shape[0], 1, D)
    Wr_b4 = Wr_b.reshape(R, Wr_b.shape[1], 1, D)
    W0_wb = W0_w.astype(jnp.bfloat16)
    Wr_wb = Wr_w.astype(jnp.bfloat16)
    ln_g2 = ln_g.reshape(1, D)
    ln_b2 = ln_b.reshape(1, D)

    L = W0_w.shape[0]
    full = lambda *shape: pl.BlockSpec(shape, lambda b: (0,) * len(shape))

    gcn2, stats = pl.pallas_call(
        _fused_kernel,
        grid=(B,),
        in_specs=[
            pl.BlockSpec((1, N, D), lambda b: (b, 0, 0)),
            pl.BlockSpec(memory_space=pltpu.MemorySpace.HBM),
            full(L, D, D),
            full(L, 1, D),
            full(R, L, D, D),
            full(R, L, 1, D),
            full(1, D),
            full(1, D),
        ],
        out_specs=[
            pl.BlockSpec((1, N, D), lambda b: (b, 0, 0)),
            pl.BlockSpec((1, N, 8), lambda b: (b, 0, 0)),
        ],
        out_shape=[
            jax.ShapeDtypeStruct((B, N, D), jnp.float32),
            jax.ShapeDtypeStruct((B, N, 8), jnp.float32),
        ],
        scratch_shapes=[
            pltpu.VMEM((3, N, N), jnp.float32),
            pltpu.VMEM((N, R * N), jnp.bfloat16),
            pltpu.VMEM((R * N, D), jnp.bfloat16),
            pltpu.SemaphoreType.DMA((3, 2)),
        ],
        compiler_params=pltpu.CompilerParams(
            dimension_semantics=("arbitrary",)),
    )(nodes, adj, W0_wb, W0_b3, Wr_wb, Wr_b4, ln_g2, ln_b2)

    masks = stats[:, :, 1].astype(jnp.int32)
    return gcn2, masks


# speedup vs baseline: 1.2006x; 1.0814x over previous
import jax
import jax.numpy as jnp
from jax.experimental import pallas as pl
from jax.experimental.pallas import tpu as pltpu


def _fused_kernel(x_ref, adj_hbm, w0w_ref, w0b_ref, wrw_ref, wrb_ref,
                  lng_ref, lnb_ref, out_ref, stats_ref,
                  abuf, adjbf_ref, hall_ref, sem):
    b = pl.program_id(0)
    n = adjbf_ref.shape[0]
    n_rel = adjbf_ref.shape[1] // n
    f32 = jnp.float32

    n_b = pl.num_programs(0)
    n_slots = abuf.shape[0]

    def slot(j):
        return jax.lax.rem(b * n_rel + j, n_slots)

    nh = abuf.shape[1] // 2

    def adj_copies(bi, j, s):
        return [pltpu.make_async_copy(
            adj_hbm.at[bi, j, pl.ds(h * nh, nh)],
            abuf.at[s, pl.ds(h * nh, nh)], sem.at[s, h]) for h in (0, 1)]

    def start_copies(bi, j, s):
        for c in adj_copies(bi, j, s):
            c.start()

    def wait_copies(bi, j, s):
        for c in adj_copies(bi, j, s):
            c.wait()

    @pl.when(b == 0)
    def _prologue():
        for j in range(n_slots):
            start_copies(0, j, j)

    xb = x_ref[0].astype(jnp.bfloat16)
    ones = jnp.ones((n, 1), dtype=jnp.bfloat16)
    d = w0w_ref.shape[1]

    s1 = jnp.dot(xb, w0w_ref[0], preferred_element_type=f32) + w0b_ref[0]
    hs = [(jnp.dot(xb, wrw_ref[j, 0], preferred_element_type=f32)
           + wrb_ref[j, 0]).astype(jnp.bfloat16) for j in range(n_rel)]
    den = jnp.ones((n, 1), dtype=f32)
    rows = []
    cols = []
    for j in range(n_rel):
        sj = slot(j)
        wait_copies(b, j, sj)
        ab = abuf[sj].astype(jnp.bfloat16)
        adjbf_ref[:, j * n:(j + 1) * n] = ab

        if j + n_slots < n_rel:
            start_copies(b, j + n_slots, sj)
        else:
            jn = j + n_slots - n_rel

            @pl.when(b + 1 < n_b)
            def _prefetch_next():
                start_copies(jnp.minimum(b + 1, n_b - 1), jn, sj)

        s1 = s1 + jnp.dot(ab, hs[j], preferred_element_type=f32)
        row = jnp.sum(ab, axis=1, keepdims=True, dtype=f32)
        cols.append(jnp.sum(ab, axis=0, keepdims=True, dtype=f32))
        den = den + row
        rows.append(row)

    cols_t = jnp.transpose(jnp.concatenate(cols, axis=0))
    msk = jnp.zeros((n, 1), dtype=f32)
    for j in range(n_rel):
        msk = msk + ((rows[j] + cols_t[:, j:j + 1]) == 0.0).astype(f32)

    y1 = jnp.maximum(s1 / den, 0.0)
    x2 = y1.astype(jnp.bfloat16)

    for j in range(n_rel):
        hall_ref[j * n:(j + 1) * n, :] = (
            jnp.dot(x2, wrw_ref[j, 1], preferred_element_type=f32)
            + wrb_ref[j, 1]).astype(jnp.bfloat16)
    s2 = jnp.dot(x2, w0w_ref[1], preferred_element_type=f32) + w0b_ref[1] \
        + jnp.dot(adjbf_ref[...], hall_ref[...],
                  preferred_element_type=f32)

    y2 = jnp.maximum(s2 / den, 0.0)
    mean = jnp.mean(y2, axis=1, keepdims=True)
    var = jnp.mean((y2 - mean) ** 2, axis=1, keepdims=True)
    yn = (y2 - mean) * jax.lax.rsqrt(var + 1e-5)
    out_ref[0] = yn * lng_ref[...] + lnb_ref[...]
    stats_ref[0] = jnp.concatenate([den, msk] + [jnp.zeros_like(den)] * 6,
                                   axis=1)


def kernel(nodes, adj, section, W0_w, W0_b, Wr_w, Wr_b, ln_g, ln_b):
    B, N, D = nodes.shape
    R = adj.shape[1]
    del section

    W0_b3 = W0_b.reshape(W0_b.shape[0], 1, D)
    Wr_b4 = Wr_b.reshape(R, Wr_b.shape[1], 1, D)
    W0_wb = W0_w.astype(jnp.bfloat16)
    Wr_wb = Wr_w.astype(jnp.bfloat16)
    ln_g2 = ln_g.reshape(1, D)
    ln_b2 = ln_b.reshape(1, D)

    L = W0_w.shape[0]
    full = lambda *shape: pl.BlockSpec(shape, lambda b: (0,) * len(shape))

    gcn2, stats = pl.pallas_call(
        _fused_kernel,
        grid=(B,),
        in_specs=[
            pl.BlockSpec((1, N, D), lambda b: (b, 0, 0)),
            pl.BlockSpec(memory_space=pltpu.MemorySpace.HBM),
            full(L, D, D),
            full(L, 1, D),
            full(R, L, D, D),
            full(R, L, 1, D),
            full(1, D),
            full(1, D),
        ],
        out_specs=[
            pl.BlockSpec((1, N, D), lambda b: (b, 0, 0)),
            pl.BlockSpec((1, N, 8), lambda b: (b, 0, 0)),
        ],
        out_shape=[
            jax.ShapeDtypeStruct((B, N, D), jnp.float32),
            jax.ShapeDtypeStruct((B, N, 8), jnp.float32),
        ],
        scratch_shapes=[
            pltpu.VMEM((3, N, N), jnp.float32),
            pltpu.VMEM((N, R * N), jnp.bfloat16),
            pltpu.VMEM((R * N, D), jnp.bfloat16),
            pltpu.SemaphoreType.DMA((3, 2)),
        ],
        compiler_params=pltpu.CompilerParams(
            dimension_semantics=("arbitrary",)),
    )(nodes, adj, W0_wb, W0_b3, Wr_wb, Wr_b4, ln_g2, ln_b2)

    masks = stats[:, :, 1].astype(jnp.int32)
    return gcn2, masks


# speedup vs baseline: 1.2548x; 1.0452x over previous
import jax
import jax.numpy as jnp
from jax.experimental import pallas as pl
from jax.experimental.pallas import tpu as pltpu


def _fused_kernel(x_ref, adj_hbm, w0w_ref, w0b_ref, wrw_ref, wrb_ref,
                  lng_ref, lnb_ref, out_ref, stats_ref,
                  abuf, adjbf_ref, hall_ref, sem):
    b = pl.program_id(0)
    n = adjbf_ref.shape[0]
    n_rel = adjbf_ref.shape[1] // n
    f32 = jnp.float32

    n_b = pl.num_programs(0)
    n_slots = abuf.shape[0]

    def slot(j):
        return jax.lax.rem(b * n_rel + j, n_slots)

    nh = abuf.shape[1] // 2

    def adj_copies(bi, j, s):
        return [pltpu.make_async_copy(
            adj_hbm.at[bi, j, pl.ds(h * nh, nh)],
            abuf.at[s, pl.ds(h * nh, nh)], sem.at[s, h]) for h in (0, 1)]

    def start_copies(bi, j, s):
        for c in adj_copies(bi, j, s):
            c.start()

    def wait_copies(bi, j, s):
        for c in adj_copies(bi, j, s):
            c.wait()

    @pl.when(b == 0)
    def _prologue():
        for j in range(n_slots):
            start_copies(0, j, j)

    xb = x_ref[0].astype(jnp.bfloat16)
    ones = jnp.ones((n, 1), dtype=jnp.bfloat16)
    d = w0w_ref.shape[1]

    s1 = jnp.dot(xb, w0w_ref[0], preferred_element_type=f32) + w0b_ref[0]
    hs = [(jnp.dot(xb, wrw_ref[j, 0], preferred_element_type=f32)
           + wrb_ref[j, 0]).astype(jnp.bfloat16) for j in range(n_rel)]
    den = jnp.ones((n, 1), dtype=f32)
    rows = []
    cols = []
    for j in range(n_rel):
        sj = slot(j)
        wait_copies(b, j, sj)
        a32 = abuf[sj]
        ab = a32.astype(jnp.bfloat16)
        adjbf_ref[:, j * n:(j + 1) * n] = ab
        row = jnp.sum(a32, axis=1, keepdims=True)
        colL = jnp.sum(a32, axis=0, keepdims=True)

        if j + n_slots < n_rel:
            start_copies(b, j + n_slots, sj)
        else:
            jn = j + n_slots - n_rel

            @pl.when(b + 1 < n_b)
            def _prefetch_next():
                start_copies(jnp.minimum(b + 1, n_b - 1), jn, sj)

        s1 = s1 + jnp.dot(ab, hs[j], preferred_element_type=f32)
        den = den + row
        rows.append(row)
        cols.append(colL)

    cols_t = jnp.transpose(jnp.concatenate(cols, axis=0))
    msk = jnp.zeros((n, 1), dtype=f32)
    for j in range(n_rel):
        msk = msk + ((rows[j] + cols_t[:, j:j + 1]) == 0.0).astype(f32)

    y1 = jnp.maximum(s1 / den, 0.0)
    x2 = y1.astype(jnp.bfloat16)

    for j in range(n_rel):
        hall_ref[j * n:(j + 1) * n, :] = (
            jnp.dot(x2, wrw_ref[j, 1], preferred_element_type=f32)
            + wrb_ref[j, 1]).astype(jnp.bfloat16)
    s2 = jnp.dot(x2, w0w_ref[1], preferred_element_type=f32) + w0b_ref[1] \
        + jnp.dot(adjbf_ref[...], hall_ref[...],
                  preferred_element_type=f32)

    y2 = jnp.maximum(s2 / den, 0.0)
    mean = jnp.mean(y2, axis=1, keepdims=True)
    var = jnp.mean((y2 - mean) ** 2, axis=1, keepdims=True)
    yn = (y2 - mean) * jax.lax.rsqrt(var + 1e-5)
    out_ref[0] = yn * lng_ref[...] + lnb_ref[...]
    stats_ref[0] = jnp.concatenate([den, msk] + [jnp.zeros_like(den)] * 6,
                                   axis=1)


def kernel(nodes, adj, section, W0_w, W0_b, Wr_w, Wr_b, ln_g, ln_b):
    B, N, D = nodes.shape
    R = adj.shape[1]
    del section

    W0_b3 = W0_b.reshape(W0_b.shape[0], 1, D)
    Wr_b4 = Wr_b.reshape(R, Wr_b.shape[1], 1, D)
    W0_wb = W0_w.astype(jnp.bfloat16)
    Wr_wb = Wr_w.astype(jnp.bfloat16)
    ln_g2 = ln_g.reshape(1, D)
    ln_b2 = ln_b.reshape(1, D)

    L = W0_w.shape[0]
    full = lambda *shape: pl.BlockSpec(shape, lambda b: (0,) * len(shape))

    gcn2, stats = pl.pallas_call(
        _fused_kernel,
        grid=(B,),
        in_specs=[
            pl.BlockSpec((1, N, D), lambda b: (b, 0, 0)),
            pl.BlockSpec(memory_space=pltpu.MemorySpace.HBM),
            full(L, D, D),
            full(L, 1, D),
            full(R, L, D, D),
            full(R, L, 1, D),
            full(1, D),
            full(1, D),
        ],
        out_specs=[
            pl.BlockSpec((1, N, D), lambda b: (b, 0, 0)),
            pl.BlockSpec((1, N, 8), lambda b: (b, 0, 0)),
        ],
        out_shape=[
            jax.ShapeDtypeStruct((B, N, D), jnp.float32),
            jax.ShapeDtypeStruct((B, N, 8), jnp.float32),
        ],
        scratch_shapes=[
            pltpu.VMEM((3, N, N), jnp.float32),
            pltpu.VMEM((N, R * N), jnp.bfloat16),
            pltpu.VMEM((R * N, D), jnp.bfloat16),
            pltpu.SemaphoreType.DMA((3, 2)),
        ],
        compiler_params=pltpu.CompilerParams(
            dimension_semantics=("arbitrary",)),
    )(nodes, adj, W0_wb, W0_b3, Wr_wb, Wr_b4, ln_g2, ln_b2)

    masks = stats[:, :, 1].astype(jnp.int32)
    return gcn2, masks


# speedup vs baseline: 1.2733x; 1.0147x over previous
import jax
import jax.numpy as jnp
from jax.experimental import pallas as pl
from jax.experimental.pallas import tpu as pltpu


def _fused_kernel(x_ref, adj_hbm, w0w_ref, w0b_ref, wrw_ref, wrb_ref,
                  lng_ref, lnb_ref, out_ref, stats_ref,
                  abuf, adjbf_ref, hall_ref, sem):
    b = pl.program_id(0)
    n = adjbf_ref.shape[0]
    n_rel = adjbf_ref.shape[1] // n
    f32 = jnp.float32

    n_b = pl.num_programs(0)
    n_slots = abuf.shape[0]

    def slot(j):
        return jax.lax.rem(b * n_rel + j, n_slots)

    nh = abuf.shape[1] // 2

    def adj_copies(bi, j, s):
        return [pltpu.make_async_copy(
            adj_hbm.at[bi, j, pl.ds(h * nh, nh)],
            abuf.at[s, pl.ds(h * nh, nh)], sem.at[s, h]) for h in (0, 1)]

    def start_copies(bi, j, s):
        for c in adj_copies(bi, j, s):
            c.start()

    def wait_copies(bi, j, s):
        for c in adj_copies(bi, j, s):
            c.wait()

    @pl.when(b == 0)
    def _prologue():
        for j in range(n_slots):
            start_copies(0, j, j)

    xb = x_ref[0].astype(jnp.bfloat16)

    s1 = jnp.dot(xb, w0w_ref[0], preferred_element_type=f32) + w0b_ref[0]
    hs = [(jnp.dot(xb, wrw_ref[j, 0], preferred_element_type=f32)
           + wrb_ref[j, 0]).astype(jnp.bfloat16) for j in range(n_rel)]
    den = jnp.ones((n, 1), dtype=f32)
    rows = []
    cols = []
    for j in range(n_rel):
        sj = slot(j)
        wait_copies(b, j, sj)
        a32 = abuf[sj]
        ab = a32.astype(jnp.bfloat16)
        adjbf_ref[:, j * n:(j + 1) * n] = ab
        row = jnp.sum(a32, axis=1, keepdims=True)
        colL = jnp.sum(a32, axis=0, keepdims=True)

        if j + n_slots < n_rel:
            start_copies(b, j + n_slots, sj)
        else:
            jn = j + n_slots - n_rel

            @pl.when(b + 1 < n_b)
            def _prefetch_next():
                start_copies(jnp.minimum(b + 1, n_b - 1), jn, sj)

        s1 = s1 + jnp.dot(ab, hs[j], preferred_element_type=f32)
        den = den + row
        rows.append(row)
        cols.append(colL)

    cols_t = jnp.transpose(jnp.concatenate(cols, axis=0))
    msk = jnp.zeros((n, 1), dtype=f32)
    for j in range(n_rel):
        msk = msk + ((rows[j] + cols_t[:, j:j + 1]) == 0.0).astype(f32)

    y1 = jnp.maximum(s1 / den, 0.0)
    x2 = y1.astype(jnp.bfloat16)

    for j in range(n_rel):
        hall_ref[j * n:(j + 1) * n, :] = (
            jnp.dot(x2, wrw_ref[j, 1], preferred_element_type=f32)
            + wrb_ref[j, 1]).astype(jnp.bfloat16)
    s2 = jnp.dot(x2, w0w_ref[1], preferred_element_type=f32) + w0b_ref[1] \
        + jnp.dot(adjbf_ref[...], hall_ref[...],
                  preferred_element_type=f32)

    y2 = jnp.maximum(s2 / den, 0.0)
    mean = jnp.mean(y2, axis=1, keepdims=True)
    var = jnp.mean((y2 - mean) ** 2, axis=1, keepdims=True)
    yn = (y2 - mean) * jax.lax.rsqrt(var + 1e-5)
    out_ref[0] = yn * lng_ref[...] + lnb_ref[...]
    stats_ref[0] = jnp.concatenate([den, msk] + [jnp.zeros_like(den)] * 6,
                                   axis=1)


def kernel(nodes, adj, section, W0_w, W0_b, Wr_w, Wr_b, ln_g, ln_b):
    B, N, D = nodes.shape
    R = adj.shape[1]
    del section

    W0_b3 = W0_b.reshape(W0_b.shape[0], 1, D)
    Wr_b4 = Wr_b.reshape(R, Wr_b.shape[1], 1, D)
    W0_wb = W0_w.astype(jnp.bfloat16)
    Wr_wb = Wr_w.astype(jnp.bfloat16)
    ln_g2 = ln_g.reshape(1, D)
    ln_b2 = ln_b.reshape(1, D)

    L = W0_w.shape[0]
    full = lambda *shape: pl.BlockSpec(shape, lambda b: (0,) * len(shape))

    gcn2, stats = pl.pallas_call(
        _fused_kernel,
        grid=(B,),
        in_specs=[
            pl.BlockSpec((1, N, D), lambda b: (b, 0, 0)),
            pl.BlockSpec(memory_space=pltpu.MemorySpace.HBM),
            full(L, D, D),
            full(L, 1, D),
            full(R, L, D, D),
            full(R, L, 1, D),
            full(1, D),
            full(1, D),
        ],
        out_specs=[
            pl.BlockSpec((1, N, D), lambda b: (b, 0, 0)),
            pl.BlockSpec((1, N, 8), lambda b: (b, 0, 0)),
        ],
        out_shape=[
            jax.ShapeDtypeStruct((B, N, D), jnp.float32),
            jax.ShapeDtypeStruct((B, N, 8), jnp.float32),
        ],
        scratch_shapes=[
            pltpu.VMEM((3, N, N), jnp.float32),
            pltpu.VMEM((N, R * N), jnp.bfloat16),
            pltpu.VMEM((R * N, D), jnp.bfloat16),
            pltpu.SemaphoreType.DMA((3, 2)),
        ],
        compiler_params=pltpu.CompilerParams(
            dimension_semantics=("arbitrary",)),
    )(nodes, adj, W0_wb, W0_b3, Wr_wb, Wr_b4, ln_g2, ln_b2)

    masks = stats[:, :, 1].astype(jnp.int32)
    return gcn2, masks
